# baseline (speedup 1.0000x reference)
import functools
import jax
import jax.numpy as jnp
from jax import lax
from jax.experimental import pallas as pl
from jax.experimental.pallas import tpu as pltpu
from jax.experimental.pallas import tpu_sc as plsc

N_USERS = 100000
N_ITEMS = 100000
K = 64
B = 16384
_WROWS = N_USERS * K // 128

_info = plsc.get_sparse_core_info()
_NC, _NS, _L = _info.num_cores, _info.num_subcores, _info.num_lanes
_NW = _NC * _NS
_BPW = B // _NW
_NG = _BPW // _L
_CH = 256


def _mf_kernel(users_hbm, items_hbm, Uw_hbm, Mw_hbm, Ub_hbm, Mb_hbm, out_hbm,
               idx_uv, idx_iv, widx_uv, widx_iv, u_rows, m_rows, ub_v, mb_v,
               out_v, sem_u, sem_m, sem_b):
    wid = lax.axis_index("s") * _NC + lax.axis_index("c")
    base = wid * _BPW

    pltpu.sync_copy(users_hbm.at[pl.ds(base, _BPW)], idx_uv)
    pltpu.sync_copy(items_hbm.at[pl.ds(base, _BPW)], idx_iv)

    cp_ub = pltpu.async_copy(Ub_hbm.at[idx_uv], ub_v, sem_b)
    cp_mb = pltpu.async_copy(Mb_hbm.at[idx_iv], mb_v, sem_b)

    def widx_body(g, _):
        gb = g * _L
        widx_uv[pl.ds(gb, _L)] = jax.lax.shift_right_logical(
            idx_uv[pl.ds(gb, _L)], 1)
        widx_iv[pl.ds(gb, _L)] = jax.lax.shift_right_logical(
            idx_iv[pl.ds(gb, _L)], 1)
        return 0

    lax.fori_loop(0, _NG, widx_body, 0)

    cp_ub.wait()
    cp_mb.wait()

    lane = lax.iota(jnp.int32, _L)

    def chunk_body(c, _):
        cb = c * _CH
        cp_u = pltpu.async_copy(Uw_hbm.at[widx_uv.at[pl.ds(cb, _CH)]],
                                u_rows, sem_u)
        cp_m = pltpu.async_copy(Mw_hbm.at[widx_iv.at[pl.ds(cb, _CH)]],
                                m_rows, sem_m)
        cp_u.wait()
        cp_m.wait()

        def group_body(g, _):
            gb = g * _L
            pu = idx_uv[pl.ds(cb + gb, _L)] & 1
            pi = idx_iv[pl.ds(cb + gb, _L)] & 1
            res = jnp.zeros((_L,), jnp.float32)
            for r in range(_L):
                j = gb + r
                hu = pu[r] * K
                hi = pi[r] * K
                acc = None
                for t in range(K // _L):
                    u = u_rows[j, pl.ds(hu + t * _L, _L)]
                    m = m_rows[j, pl.ds(hi + t * _L, _L)]
                    p = u * m
                    acc = p if acc is None else acc + p
                s = jnp.sum(acc)
                res = jnp.where(lane == r, s, res)
            res = res + ub_v[pl.ds(cb + gb, _L)] + mb_v[pl.ds(cb + gb, _L)]
            out_v[pl.ds(cb + gb, _L)] = res
            return 0

        lax.fori_loop(0, _CH // _L, group_body, 0)
        return 0

    lax.fori_loop(0, _BPW // _CH, chunk_body, 0)

    pltpu.sync_copy(out_v, out_hbm.at[pl.ds(base, _BPW)])


@jax.jit
def _run(users, items, Uw, Mw, Ub, Mb):
    mesh = plsc.VectorSubcoreMesh(core_axis_name="c", subcore_axis_name="s")
    kfn = functools.partial(
        pl.kernel,
        out_type=jax.ShapeDtypeStruct((B,), jnp.float32),
        mesh=mesh,
        scratch_types=[
            pltpu.VMEM((_BPW,), jnp.int32),
            pltpu.VMEM((_BPW,), jnp.int32),
            pltpu.VMEM((_BPW,), jnp.int32),
            pltpu.VMEM((_BPW,), jnp.int32),
            pltpu.VMEM((_CH, 2 * K), jnp.float32),
            pltpu.VMEM((_CH, 2 * K), jnp.float32),
            pltpu.VMEM((_BPW,), jnp.float32),
            pltpu.VMEM((_BPW,), jnp.float32),
            pltpu.VMEM((_BPW,), jnp.float32),
            pltpu.SemaphoreType.DMA,
            pltpu.SemaphoreType.DMA,
            pltpu.SemaphoreType.DMA,
        ],
        compiler_params=pltpu.CompilerParams(needs_layout_passes=False),
    )(_mf_kernel)
    return kfn(users, items, Uw, Mw, Ub, Mb)


def kernel(users, items, movie_map, U, M, Ub, Mb):
    del movie_map
    return _run(users.astype(jnp.int32), items.astype(jnp.int32),
                U.reshape(_WROWS, 128), M.reshape(_WROWS, 128),
                Ub.reshape(-1), Mb.reshape(-1))
